# Initial kernel scaffold; baseline (speedup 1.0000x reference)
import jax
import jax.numpy as jnp
from jax.experimental import pallas as pl


def kernel(relation, head, entity_embedding, relation_embedding):
    raise NotImplementedError("write your pallas kernel here")


# speedup vs baseline: 4.4634x; 4.4634x over previous
import functools

import jax
import jax.numpy as jnp
from jax.experimental import pallas as pl
from jax.experimental.pallas import tpu as pltpu

N_ENT = 100000
D = 256
B = 1024
K = 10
W = 4096
NW = (N_ENT + W - 1) // W

NEG = -3.0e38


def _topk_kernel(q_ref, e_ref, ov_ref, oi_ref, vs_ref, is_ref):
    w = pl.program_id(0)

    @pl.when(w == 0)
    def _():
        vs_ref[...] = jnp.full((B, 16), NEG, jnp.float32)
        is_ref[...] = jnp.zeros((B, 16), jnp.int32)

    scores = jax.lax.dot_general(
        q_ref[...], e_ref[...],
        (((1,), (1,)), ((), ())),
        preferred_element_type=jnp.float32,
    )
    gidx = w * W + jax.lax.broadcasted_iota(jnp.int32, (B, W), 1)
    scores = jnp.where(gidx < N_ENT, scores, NEG)

    allv = jnp.concatenate([scores, vs_ref[...]], axis=1)
    alli = jnp.concatenate([gidx, is_ref[...]], axis=1)
    pos = jax.lax.broadcasted_iota(jnp.int32, (B, W + 16), 1)

    new_v = []
    new_i = []
    for _ in range(K):
        m = jnp.max(allv, axis=1, keepdims=True)
        eq = allv == m
        p = jnp.min(jnp.where(eq, pos, jnp.int32(2**30)), axis=1, keepdims=True)
        sel = pos == p
        it = jnp.sum(jnp.where(sel, alli, 0), axis=1, keepdims=True)
        new_v.append(m)
        new_i.append(it)
        allv = jnp.where(sel, NEG, allv)

    pad_v = jnp.full((B, 16 - K), NEG, jnp.float32)
    pad_i = jnp.zeros((B, 16 - K), jnp.int32)
    vs_ref[...] = jnp.concatenate(new_v + [pad_v], axis=1)
    is_ref[...] = jnp.concatenate(new_i + [pad_i], axis=1)

    @pl.when(w == NW - 1)
    def _():
        ov_ref[...] = vs_ref[...]
        oi_ref[...] = is_ref[...]


@functools.partial(jax.jit, static_argnames=("interpret",))
def kernel(relation, head, entity_embedding, relation_embedding, interpret=False):
    relation = relation.reshape(-1)
    head = head.reshape(-1)
    head_emb = jnp.take(entity_embedding, head, axis=0)
    rel_emb = jnp.take(relation_embedding, relation, axis=0)
    query = head_emb * rel_emb

    vals, idxs = pl.pallas_call(
        _topk_kernel,
        grid=(NW,),
        in_specs=[
            pl.BlockSpec((B, D), lambda w: (0, 0)),
            pl.BlockSpec((W, D), lambda w: (w, 0)),
        ],
        out_specs=[
            pl.BlockSpec((B, 16), lambda w: (0, 0)),
            pl.BlockSpec((B, 16), lambda w: (0, 0)),
        ],
        out_shape=[
            jax.ShapeDtypeStruct((B, 16), jnp.float32),
            jax.ShapeDtypeStruct((B, 16), jnp.int32),
        ],
        scratch_shapes=[
            pltpu.VMEM((B, 16), jnp.float32),
            pltpu.VMEM((B, 16), jnp.int32),
        ],
        interpret=interpret,
    )(query, entity_embedding)

    return vals[:, :K], idxs[:, :K]


# speedup vs baseline: 12.5638x; 2.8149x over previous
import functools

import jax
import jax.numpy as jnp
from jax import lax
from jax.experimental import pallas as pl
from jax.experimental.pallas import tpu as pltpu
from jax.experimental.pallas import tpu_sc as plsc

N_ENT = 100000
D = 256
B = 1024
K = 10
W = 4096
NW = 25
NPAD = NW * W
CHUNK = 128
CPW = W // CHUNK
NCHUNK = NW * CPW
NSEL = K
NEG = -3.0e38
BIG = 2**30

SC_CORES = 2
SC_SUBCORES = 16
SC_TILES = SC_CORES * SC_SUBCORES

_INTERPRET = False


def _score_kernel(h_ref, r_ref, e_ref, s_ref, cm_ref, q_ref):
    w = pl.program_id(0)

    @pl.when(w == 0)
    def _():
        q_ref[...] = h_ref[...] * r_ref[...]

    q = q_ref[...]

    def pieces(masked):
        def go():
            maxes = []
            for c in range(CPW):
                eb = e_ref[pl.ds(c * CHUNK, CHUNK), :]
                sc = lax.dot_general(
                    q, eb, (((1,), (1,)), ((), ())),
                    preferred_element_type=jnp.float32,
                )
                if masked:
                    gidx = w * W + c * CHUNK + lax.broadcasted_iota(
                        jnp.int32, (B, CHUNK), 1)
                    sc = jnp.where(gidx < N_ENT, sc, NEG)
                s_ref[:, c * CHUNK:(c + 1) * CHUNK] = sc
                maxes.append(jnp.max(sc, axis=1, keepdims=True))
            cm_ref[0] = jnp.concatenate(maxes, axis=1)
        return go

    pl.when(w < NW - 1)(pieces(False))
    pl.when(w == NW - 1)(pieces(True))


def _score_pallas(h_emb, r_emb, entity_embedding):
    return pl.pallas_call(
        _score_kernel,
        grid=(NW,),
        in_specs=[
            pl.BlockSpec((B, D), lambda w: (0, 0)),
            pl.BlockSpec((B, D), lambda w: (0, 0)),
            pl.BlockSpec((W, D), lambda w: (w, 0)),
        ],
        out_specs=[
            pl.BlockSpec((B, W), lambda w: (0, w)),
            pl.BlockSpec((1, B, CPW), lambda w: (w, 0, 0)),
        ],
        out_shape=[
            jax.ShapeDtypeStruct((B, NPAD), jnp.float32),
            jax.ShapeDtypeStruct((NW, B, CPW), jnp.float32),
        ],
        scratch_shapes=[pltpu.VMEM((B, D), jnp.float32)],
        interpret=_INTERPRET,
    )(h_emb, r_emb, entity_embedding)


def _select_kernel(cm_ref, f_ref):
    allv = cm_ref[...]
    pos = lax.broadcasted_iota(jnp.int32, (B, NCHUNK), 1)
    row = lax.broadcasted_iota(jnp.int32, (B, NSEL), 0)
    cols = []
    for _ in range(NSEL):
        m = jnp.max(allv, axis=1, keepdims=True)
        eq = allv == m
        p = jnp.min(jnp.where(eq, pos, BIG), axis=1, keepdims=True)
        cols.append(p)
        allv = jnp.where(pos == p, NEG, allv)
    cid = jnp.concatenate(cols, axis=1)
    flat = row * NCHUNK + cid
    f_ref[...] = jnp.concatenate(
        [flat, jnp.zeros((B, 16 - NSEL), jnp.int32)], axis=1)


def _select_pallas(cmax):
    return pl.pallas_call(
        _select_kernel,
        out_shape=jax.ShapeDtypeStruct((B, 16), jnp.int32),
        interpret=_INTERPRET,
    )(cmax)


def _final_kernel(c_ref, f_ref, ov_ref, oi_ref):
    allv = c_ref[...]
    flat = f_ref[...][:, :NSEL]
    row = lax.broadcasted_iota(jnp.int32, (B, NSEL), 0)
    cid = flat - row * NCHUNK
    lane = lax.broadcasted_iota(jnp.int32, (B, NSEL, CHUNK), 2)
    ent = cid[:, :, None] * CHUNK + lane
    vs, idxs = [], []
    for _ in range(K):
        m = jnp.max(jnp.max(allv, axis=2), axis=1, keepdims=True)
        eq = allv == m[:, :, None]
        e = jnp.min(jnp.min(jnp.where(eq, ent, BIG), axis=2), axis=1,
                    keepdims=True)
        sel = eq & (ent == e[:, :, None])
        vs.append(m)
        idxs.append(e)
        allv = jnp.where(sel, NEG, allv)
    pad_v = jnp.full((B, 16 - K), NEG, jnp.float32)
    pad_i = jnp.zeros((B, 16 - K), jnp.int32)
    ov_ref[...] = jnp.concatenate(vs + [pad_v], axis=1)
    oi_ref[...] = jnp.concatenate(idxs + [pad_i], axis=1)


def _final_pallas(cand, flat16):
    return pl.pallas_call(
        _final_kernel,
        out_shape=[
            jax.ShapeDtypeStruct((B, 16), jnp.float32),
            jax.ShapeDtypeStruct((B, 16), jnp.int32),
        ],
        interpret=_INTERPRET,
    )(cand, flat16)


def _gather_query(entity_embedding, relation_embedding, head_idx, rel_idx):
    per = B // SC_TILES
    mesh = plsc.VectorSubcoreMesh(core_axis_name="c", subcore_axis_name="s")

    @functools.partial(
        pl.kernel, mesh=mesh,
        out_type=[
            jax.ShapeDtypeStruct((B, D), jnp.float32),
            jax.ShapeDtypeStruct((B, D), jnp.float32),
        ],
        scratch_types=[
            pltpu.VMEM((per,), jnp.int32),
            pltpu.VMEM((per,), jnp.int32),
            pltpu.VMEM((per, D), jnp.float32),
            pltpu.VMEM((per, D), jnp.float32),
            pltpu.SemaphoreType.DMA,
        ],
    )
    def gq(ent_hbm, rel_hbm, hi_hbm, ri_hbm, oh_hbm, orr_hbm,
           hi_v, ri_v, hr_v, rr_v, sem):
        wid = lax.axis_index("s") * SC_CORES + lax.axis_index("c")
        base = wid * per
        pltpu.sync_copy(hi_hbm.at[pl.ds(base, per)], hi_v)
        pltpu.sync_copy(ri_hbm.at[pl.ds(base, per)], ri_v)
        pltpu.async_copy(ent_hbm.at[hi_v], hr_v, sem).wait()
        pltpu.async_copy(rel_hbm.at[ri_v], rr_v, sem).wait()
        pltpu.sync_copy(hr_v, oh_hbm.at[pl.ds(base, per)])
        pltpu.sync_copy(rr_v, orr_hbm.at[pl.ds(base, per)])

    return gq(entity_embedding, relation_embedding, head_idx, rel_idx)


def _gather_cand(score_rows, flat_idx):
    total = B * NSEL
    per = total // SC_TILES
    sub = 80
    mesh = plsc.VectorSubcoreMesh(core_axis_name="c", subcore_axis_name="s")

    @functools.partial(
        pl.kernel, mesh=mesh,
        out_type=jax.ShapeDtypeStruct((total, CHUNK), jnp.float32),
        scratch_types=[
            pltpu.VMEM((per,), jnp.int32),
            pltpu.VMEM((per, CHUNK), jnp.float32),
            pltpu.SemaphoreType.DMA,
        ],
    )
    def gc(rows_hbm, idx_hbm, out_hbm, idx_v, rows_v, sem):
        wid = lax.axis_index("s") * SC_CORES + lax.axis_index("c")
        base = wid * per
        pltpu.sync_copy(idx_hbm.at[pl.ds(base, per)], idx_v)
        cps = [
            pltpu.async_copy(
                rows_hbm.at[idx_v.at[pl.ds(j * sub, sub)]],
                rows_v.at[pl.ds(j * sub, sub)], sem)
            for j in range(per // sub)
        ]
        for cp in cps:
            cp.wait()
        pltpu.sync_copy(rows_v, out_hbm.at[pl.ds(base, per)])

    return gc(score_rows, flat_idx)


@jax.jit
def kernel(relation, head, entity_embedding, relation_embedding):
    head_idx = head.reshape(-1).astype(jnp.int32)
    rel_idx = relation.reshape(-1).astype(jnp.int32)

    h_emb, r_emb = _gather_query(
        entity_embedding, relation_embedding, head_idx, rel_idx)
    scores, cmax3 = _score_pallas(h_emb, r_emb, entity_embedding)
    cmax = cmax3.transpose(1, 0, 2).reshape(B, NCHUNK)
    flat16 = _select_pallas(cmax)
    flat = flat16[:, :NSEL].reshape(B * NSEL)
    cand = _gather_cand(scores.reshape(B * NPAD // CHUNK, CHUNK), flat)
    vals16, idx16 = _final_pallas(cand.reshape(B, NSEL, CHUNK), flat16)
    return vals16[:, :K], idx16[:, :K]
